# Initial kernel scaffold; baseline (speedup 1.0000x reference)
import jax
import jax.numpy as jnp
from jax.experimental import pallas as pl


def kernel(Graph, norm_h, norm_L, norm_W, norm_P, norm_N, T, ln_g, ln_b, W0, We0, al0, ar0, ae0, b0, W1, We1, al1, ar1, ae1, b1, Wl, bl):
    raise NotImplementedError("write your pallas kernel here")


# speedup vs baseline: 3.9343x; 3.9343x over previous
import jax
import jax.numpy as jnp
from jax.experimental import pallas as pl

J = 100
M = 20
N = J + M
NP = 128
BS = 256
H = 3
F0 = 16
ED = 128
BB = 8
K2 = NP * ED
KB = 2048
NK = K2 // KB


def _lrelu(x, s):
    return jnp.where(x >= 0, x, s * x)


def _mm(x3, w):
    b, n, k = x3.shape
    y = jnp.dot(x3.reshape(b * n, k), w, preferred_element_type=jnp.float32)
    return y.reshape(b, n, -1)


def _gnn_block(x_ref, a_ref, t_ref, w0_ref, g_ref, bln_ref,
               al0_ref, ar0_ref, few0_ref, ae0_ref, b0_ref,
               w1_ref, al1_ref, ar1_ref, few1_ref, ae1_ref, b1_ref,
               h1_ref):
    X = x_ref[...]
    lane = jax.lax.broadcasted_iota(jnp.int32, X.shape, 2)
    m5 = (lane < 5).astype(jnp.float32)
    mu = jnp.sum(X, axis=-1, keepdims=True) * (1.0 / 5.0)
    d = (X - mu) * m5
    var = jnp.sum(d * d, axis=-1, keepdims=True) * (1.0 / 5.0)
    Xn = d * jax.lax.rsqrt(var + 1e-5)
    Xn = Xn * g_ref[...].reshape(1, 1, 8) + bln_ref[...].reshape(1, 1, 8)

    Ab = a_ref[...] > 0
    Tm = t_ref[...]

    def gat(ft, al_ref, ar_ref, few_ref, ae_ref, b_ref, Fh):
        acc = None
        for h in range(H):
            fth = ft[:, :, h * Fh:(h + 1) * Fh]
            alh = al_ref[h:h + 1, :].reshape(1, 1, Fh)
            arh = ar_ref[h:h + 1, :].reshape(1, 1, Fh)
            fewh = few_ref[h:h + 1, :]
            eec = jnp.sum(fewh * ae_ref[h:h + 1, :])
            el = jnp.sum(fth * alh, axis=-1)
            er = jnp.sum(fth * arh, axis=-1)
            logits = el[:, :, None] + er[:, None, :] + Tm * eec
            logits = _lrelu(logits, 0.2)
            logits = jnp.where(Ab, logits, -1e9)
            mx = jnp.max(logits, axis=1, keepdims=True)
            ex = jnp.where(Ab, jnp.exp(logits - mx), 0.0)
            den = jnp.sum(ex, axis=1, keepdims=True)
            alpha = ex / jnp.where(den > 0, den, 1.0)
            outh = jax.lax.dot_general(
                alpha, fth, (((1,), (1,)), ((0,), (0,))),
                preferred_element_type=jnp.float32)
            eagg = jnp.sum(alpha * Tm, axis=1)
            hh = outh + eagg[:, :, None] * fewh.reshape(1, 1, Fh) \
                + b_ref[h:h + 1, :].reshape(1, 1, Fh)
            hh = _lrelu(hh, 0.01)
            acc = hh if acc is None else acc + hh
        return acc * (1.0 / H)

    ft0 = _mm(Xn, w0_ref[...])
    h0 = gat(ft0, al0_ref, ar0_ref, few0_ref, ae0_ref, b0_ref, F0)
    ft1 = _mm(h0, w1_ref[...])
    h1_ref[...] = gat(ft1, al1_ref, ar1_ref, few1_ref, ae1_ref, b1_ref, ED)


def _fin_block(h_ref, w_ref, b_ref, o_ref):
    k = pl.program_id(0)
    part = jnp.dot(h_ref[...], w_ref[...], preferred_element_type=jnp.float32)

    @pl.when(k == 0)
    def _():
        o_ref[...] = part

    @pl.when(k > 0)
    def _():
        o_ref[...] += part

    @pl.when(k == NK - 1)
    def _():
        o_ref[...] = _lrelu(o_ref[...] + b_ref[...], 0.01)


def kernel(Graph, norm_h, norm_L, norm_W, norm_P, norm_N, T, ln_g, ln_b,
           W0, We0, al0, ar0, ae0, b0, W1, We1, al1, ar1, ae1, b1, Wl, bl):
    G3 = Graph.reshape(BS, J, N)
    Ap = jnp.pad(G3, ((0, 0), (0, NP - J), (0, NP - N)))
    Tp = jnp.pad(T, ((0, 0), (0, NP - J), (0, NP - J)))
    other = jnp.concatenate([norm_W, norm_P, norm_N], axis=1)
    jobs = jnp.stack([norm_h, norm_L], axis=-1)
    jobf = jnp.concatenate(
        [jobs, jnp.broadcast_to(other[:, None, :], (BS, J, 3))], axis=-1)
    X = jnp.pad(jobf, ((0, 0), (0, NP - J), (0, 3)))
    g8 = jnp.pad(ln_g, (0, 3)).reshape(1, 8)
    b8 = jnp.pad(ln_b, (0, 3)).reshape(1, 8)
    W0p = jnp.pad(W0, ((0, 3), (0, 0)))
    few0 = We0.reshape(H, F0)
    few1 = We1.reshape(H, ED)
    b0m = b0.reshape(H, F0)
    b1m = b1.reshape(H, ED)
    Wl3 = jnp.pad(Wl.reshape(N, ED, ED),
                  ((0, NP - N), (0, 0), (0, 0))).reshape(K2, ED)
    blr = bl.reshape(1, ED)

    rep2 = lambda i: (0, 0)
    h1 = pl.pallas_call(
        _gnn_block,
        grid=(BS // BB,),
        in_specs=[
            pl.BlockSpec((BB, NP, 8), lambda i: (i, 0, 0)),
            pl.BlockSpec((BB, NP, NP), lambda i: (i, 0, 0)),
            pl.BlockSpec((BB, NP, NP), lambda i: (i, 0, 0)),
            pl.BlockSpec((8, H * F0), rep2),
            pl.BlockSpec((1, 8), rep2),
            pl.BlockSpec((1, 8), rep2),
            pl.BlockSpec((H, F0), rep2),
            pl.BlockSpec((H, F0), rep2),
            pl.BlockSpec((H, F0), rep2),
            pl.BlockSpec((H, F0), rep2),
            pl.BlockSpec((H, F0), rep2),
            pl.BlockSpec((F0, H * ED), rep2),
            pl.BlockSpec((H, ED), rep2),
            pl.BlockSpec((H, ED), rep2),
            pl.BlockSpec((H, ED), rep2),
            pl.BlockSpec((H, ED), rep2),
            pl.BlockSpec((H, ED), rep2),
        ],
        out_specs=pl.BlockSpec((BB, NP, ED), lambda i: (i, 0, 0)),
        out_shape=jax.ShapeDtypeStruct((BS, NP, ED), jnp.float32),
    )(X, Ap, Tp, W0p, g8, b8, al0, ar0, few0, ae0, b0m,
      W1, al1, ar1, few1, ae1, b1m)

    out = pl.pallas_call(
        _fin_block,
        grid=(NK,),
        in_specs=[
            pl.BlockSpec((BS, KB), lambda k: (0, k)),
            pl.BlockSpec((KB, ED), lambda k: (k, 0)),
            pl.BlockSpec((1, ED), lambda k: (0, 0)),
        ],
        out_specs=pl.BlockSpec((BS, ED), lambda k: (0, 0)),
        out_shape=jax.ShapeDtypeStruct((BS, ED), jnp.float32),
    )(h1.reshape(BS, K2), Wl3, blr)
    return out


# speedup vs baseline: 4.3466x; 1.1048x over previous
import jax
import jax.numpy as jnp
from jax.experimental import pallas as pl

J = 100
M = 20
N = J + M
NP = 128
BS = 256
H = 3
F0 = 16
ED = 128
BB = 8
K2 = NP * ED
KB = 2048
NK = K2 // KB


def _lrelu(x, s):
    return jnp.maximum(x, s * x)


def _mm(x3, w):
    b, n, k = x3.shape
    y = jnp.dot(x3.reshape(b * n, k), w, preferred_element_type=jnp.float32)
    return y.reshape(b, n, -1)


def _gnn_block(x_ref, a_ref, t_ref, w0_ref, g_ref, bln_ref,
               al0_ref, ar0_ref, few0_ref, ae0_ref, b0_ref,
               w1_ref, al1_ref, ar1_ref, few1_ref, ae1_ref, b1_ref,
               h1_ref):
    X = x_ref[...]
    lane = jax.lax.broadcasted_iota(jnp.int32, X.shape, 2)
    m5 = (lane < 5).astype(jnp.float32)
    mu = jnp.sum(X, axis=-1, keepdims=True) * (1.0 / 5.0)
    d = (X - mu) * m5
    var = jnp.sum(d * d, axis=-1, keepdims=True) * (1.0 / 5.0)
    Xn = d * jax.lax.rsqrt(var + 1e-5)
    Xn = Xn * g_ref[...].reshape(1, 1, 8) + bln_ref[...].reshape(1, 1, 8)

    G = a_ref[...]
    zc = jnp.zeros((BB, NP - J, N), jnp.float32)
    Gp = jnp.concatenate([G, zc], axis=1)
    Gp = jnp.concatenate([Gp, jnp.zeros((BB, NP, NP - N), jnp.float32)], axis=2)
    Ab = Gp > 0
    Tr = t_ref[...]
    Tm = jnp.concatenate([Tr, jnp.zeros((BB, NP - J, J), jnp.float32)], axis=1)
    Tm = jnp.concatenate([Tm, jnp.zeros((BB, NP, NP - J), jnp.float32)], axis=2)

    def gat(ft, al_ref, ar_ref, few_ref, ae_ref, b_ref, Fh):
        acc = None
        for h in range(H):
            fth = ft[:, :, h * Fh:(h + 1) * Fh]
            alh = al_ref[h:h + 1, :].reshape(1, 1, Fh)
            arh = ar_ref[h:h + 1, :].reshape(1, 1, Fh)
            fewh = few_ref[h:h + 1, :]
            eec = jnp.sum(fewh * ae_ref[h:h + 1, :])
            el = jnp.sum(fth * alh, axis=-1)
            er = jnp.sum(fth * arh, axis=-1)
            logits = el[:, :, None] + er[:, None, :] + Tm * eec
            logits = _lrelu(logits, 0.2)
            logits = jnp.where(Ab, logits, -1e9)
            mx = jnp.max(logits, axis=1, keepdims=True)
            ex = jnp.where(Ab, jnp.exp(logits - mx), 0.0)
            den = jnp.sum(ex, axis=1, keepdims=True)
            alpha = ex / jnp.where(den > 0, den, 1.0)
            outh = jax.lax.dot_general(
                alpha, fth, (((1,), (1,)), ((0,), (0,))),
                preferred_element_type=jnp.float32)
            eagg = jnp.sum(alpha * Tm, axis=1)
            hh = outh + eagg[:, :, None] * fewh.reshape(1, 1, Fh) \
                + b_ref[h:h + 1, :].reshape(1, 1, Fh)
            hh = _lrelu(hh, 0.01)
            acc = hh if acc is None else acc + hh
        return acc * (1.0 / H)

    ft0 = _mm(Xn, w0_ref[...])
    h0 = gat(ft0, al0_ref, ar0_ref, few0_ref, ae0_ref, b0_ref, F0)
    ft1 = _mm(h0, w1_ref[...])
    h1_ref[...] = gat(ft1, al1_ref, ar1_ref, few1_ref, ae1_ref, b1_ref, ED)


def _fin_block(h_ref, w_ref, b_ref, o_ref):
    k = pl.program_id(0)
    part = jnp.dot(h_ref[...], w_ref[...], preferred_element_type=jnp.float32)

    @pl.when(k == 0)
    def _():
        o_ref[...] = part

    @pl.when(k > 0)
    def _():
        o_ref[...] += part

    @pl.when(k == NK - 1)
    def _():
        o_ref[...] = _lrelu(o_ref[...] + b_ref[...], 0.01)


def kernel(Graph, norm_h, norm_L, norm_W, norm_P, norm_N, T, ln_g, ln_b,
           W0, We0, al0, ar0, ae0, b0, W1, We1, al1, ar1, ae1, b1, Wl, bl):
    G3 = Graph.reshape(BS, J, N)
    other = jnp.concatenate([norm_W, norm_P, norm_N], axis=1)
    jobs = jnp.stack([norm_h, norm_L], axis=-1)
    jobf = jnp.concatenate(
        [jobs, jnp.broadcast_to(other[:, None, :], (BS, J, 3))], axis=-1)
    X = jnp.pad(jobf, ((0, 0), (0, NP - J), (0, 3)))
    g8 = jnp.pad(ln_g, (0, 3)).reshape(1, 8)
    b8 = jnp.pad(ln_b, (0, 3)).reshape(1, 8)
    W0p = jnp.pad(W0, ((0, 3), (0, 0)))
    few0 = We0.reshape(H, F0)
    few1 = We1.reshape(H, ED)
    b0m = b0.reshape(H, F0)
    b1m = b1.reshape(H, ED)
    Wl3 = jnp.pad(Wl.reshape(N, ED, ED),
                  ((0, NP - N), (0, 0), (0, 0))).reshape(K2, ED)
    blr = bl.reshape(1, ED)

    rep2 = lambda i: (0, 0)
    h1 = pl.pallas_call(
        _gnn_block,
        grid=(BS // BB,),
        in_specs=[
            pl.BlockSpec((BB, NP, 8), lambda i: (i, 0, 0)),
            pl.BlockSpec((BB, J, N), lambda i: (i, 0, 0)),
            pl.BlockSpec((BB, J, J), lambda i: (i, 0, 0)),
            pl.BlockSpec((8, H * F0), rep2),
            pl.BlockSpec((1, 8), rep2),
            pl.BlockSpec((1, 8), rep2),
            pl.BlockSpec((H, F0), rep2),
            pl.BlockSpec((H, F0), rep2),
            pl.BlockSpec((H, F0), rep2),
            pl.BlockSpec((H, F0), rep2),
            pl.BlockSpec((H, F0), rep2),
            pl.BlockSpec((F0, H * ED), rep2),
            pl.BlockSpec((H, ED), rep2),
            pl.BlockSpec((H, ED), rep2),
            pl.BlockSpec((H, ED), rep2),
            pl.BlockSpec((H, ED), rep2),
            pl.BlockSpec((H, ED), rep2),
        ],
        out_specs=pl.BlockSpec((BB, NP, ED), lambda i: (i, 0, 0)),
        out_shape=jax.ShapeDtypeStruct((BS, NP, ED), jnp.float32),
    )(X, G3, T, W0p, g8, b8, al0, ar0, few0, ae0, b0m,
      W1, al1, ar1, few1, ae1, b1m)

    out = pl.pallas_call(
        _fin_block,
        grid=(NK,),
        in_specs=[
            pl.BlockSpec((BS, KB), lambda k: (0, k)),
            pl.BlockSpec((KB, ED), lambda k: (k, 0)),
            pl.BlockSpec((1, ED), lambda k: (0, 0)),
        ],
        out_specs=pl.BlockSpec((BS, ED), lambda k: (0, 0)),
        out_shape=jax.ShapeDtypeStruct((BS, ED), jnp.float32),
    )(h1.reshape(BS, K2), Wl3, blr)
    return out


# speedup vs baseline: 6.5474x; 1.5063x over previous
import jax
import jax.numpy as jnp
from jax.experimental import pallas as pl

J = 100
M = 20
N = J + M
NP = 128
BS = 256
H = 3
F0 = 16
ED = 128
BB = 8
K2 = NP * ED
KB = 2048
NK = K2 // KB


def _lrelu(x, s):
    return jnp.maximum(x, s * x)


def _mm(x3, w):
    b, n, k = x3.shape
    y = jnp.dot(x3.reshape(b * n, k), w, preferred_element_type=jnp.float32)
    return y.reshape(b, n, -1)


def _gnn_block(xt_ref, a_ref, t_ref, w0_ref, g_ref, bln_ref,
               wer0_ref, few0t_ref, ae0t_ref, b0t_ref,
               w1_ref, wer1_ref, few1t_ref, ae1t_ref, b1t_ref,
               h1_ref):
    XT = xt_ref[...]
    row = jax.lax.broadcasted_iota(jnp.int32, XT.shape, 1)
    m5 = (row < 5).astype(jnp.float32)
    mu = jnp.sum(XT, axis=1, keepdims=True) * (1.0 / 5.0)
    d = (XT - mu) * m5
    var = jnp.sum(d * d, axis=1, keepdims=True) * (1.0 / 5.0)
    XnT = d * jax.lax.rsqrt(var + 1e-5) * g_ref[...].reshape(1, 8, 1) \
        + bln_ref[...].reshape(1, 8, 1)

    G = a_ref[...]
    Gp = jnp.concatenate([G, jnp.zeros((BB, NP - J, N), jnp.float32)], axis=1)
    Gp = jnp.concatenate([Gp, jnp.zeros((BB, NP, NP - N), jnp.float32)], axis=2)
    Ab = Gp > 0
    Tr = t_ref[...]
    Tm = jnp.concatenate([Tr, jnp.zeros((BB, NP - J, J), jnp.float32)], axis=1)
    Tm = jnp.concatenate([Tm, jnp.zeros((BB, NP, NP - J), jnp.float32)], axis=2)

    def gat_T(ftaug, srcT, elbase, wer_ref, fewt_ref, aet_ref, bt_ref, Fh):
        C = srcT.shape[1]
        acc = None
        for h in range(H):
            fth = ftaug[:, :, h * Fh:(h + 1) * Fh]
            el = ftaug[:, :, elbase + h:elbase + h + 1]
            er = jnp.sum(srcT * wer_ref[:, h:h + 1].reshape(1, C, 1),
                         axis=1, keepdims=True)
            eec = jnp.sum(fewt_ref[:, h:h + 1] * aet_ref[:, h:h + 1])
            logits = el + er + Tm * eec
            logits = _lrelu(logits, 0.2)
            logits = jnp.where(Ab, logits, -1e9)
            mx = jnp.max(logits, axis=1, keepdims=True)
            ex = jnp.where(Ab, jnp.exp(logits - mx), 0.0)
            den = jnp.sum(ex, axis=1, keepdims=True)
            rden = jnp.where(den > 0, 1.0 / den, 0.0)
            outT = jax.lax.dot_general(
                fth, ex, (((1,), (1,)), ((0,), (0,))),
                preferred_element_type=jnp.float32)
            eaggT = jnp.sum(ex * Tm, axis=1, keepdims=True)
            fewcol = fewt_ref[:, h:h + 1].reshape(1, Fh, 1)
            bcol = bt_ref[:, h:h + 1].reshape(1, Fh, 1)
            hh = _lrelu((outT + eaggT * fewcol) * rden + bcol, 0.01)
            acc = hh if acc is None else acc + hh
        return acc * (1.0 / H)

    Xn = jnp.swapaxes(XnT, 1, 2)
    ft0aug = _mm(Xn, w0_ref[...])
    h0T = gat_T(ft0aug, XnT, 3 * F0, wer0_ref, few0t_ref, ae0t_ref,
                b0t_ref, F0)
    h0 = jnp.swapaxes(h0T, 1, 2)
    ft1aug = _mm(h0, w1_ref[...])
    h1_ref[...] = gat_T(ft1aug, h0T, H * ED, wer1_ref, few1t_ref, ae1t_ref,
                        b1t_ref, ED)


def _fin_block(h_ref, w_ref, b_ref, o_ref):
    k = pl.program_id(0)
    part = jnp.dot(h_ref[...], w_ref[...], preferred_element_type=jnp.float32)

    @pl.when(k == 0)
    def _():
        o_ref[...] = part

    @pl.when(k > 0)
    def _():
        o_ref[...] += part

    @pl.when(k == NK - 1)
    def _():
        o_ref[...] = _lrelu(o_ref[...] + b_ref[...], 0.01)


def kernel(Graph, norm_h, norm_L, norm_W, norm_P, norm_N, T, ln_g, ln_b,
           W0, We0, al0, ar0, ae0, b0, W1, We1, al1, ar1, ae1, b1, Wl, bl):
    G3 = Graph.reshape(BS, J, N)
    zpad = jnp.zeros((BS, NP - J), jnp.float32)
    zrow = jnp.zeros((BS, NP), jnp.float32)
    rows = [jnp.concatenate([norm_h, zpad], axis=1),
            jnp.concatenate([norm_L, zpad], axis=1),
            jnp.concatenate([jnp.broadcast_to(norm_W, (BS, J)), zpad], axis=1),
            jnp.concatenate([jnp.broadcast_to(norm_P, (BS, J)), zpad], axis=1),
            jnp.concatenate([jnp.broadcast_to(norm_N, (BS, J)), zpad], axis=1),
            zrow, zrow, zrow]
    XT = jnp.stack(rows, axis=1)

    g8 = jnp.pad(ln_g, (0, 3)).reshape(8, 1)
    b8 = jnp.pad(ln_b, (0, 3)).reshape(8, 1)

    W0r = W0.reshape(5, H, F0)
    wel0 = jnp.einsum('chf,hf->ch', W0r, al0)
    wer0 = jnp.pad(jnp.einsum('chf,hf->ch', W0r, ar0), ((0, 3), (0, 0)))
    W0aug = jnp.concatenate(
        [jnp.pad(W0, ((0, 3), (0, 0))),
         jnp.pad(wel0, ((0, 3), (0, F0 - H)))], axis=1)
    W1r = W1.reshape(F0, H, ED)
    wel1 = jnp.einsum('chf,hf->ch', W1r, al1)
    wer1 = jnp.einsum('chf,hf->ch', W1r, ar1)
    W1aug = jnp.concatenate(
        [W1, jnp.pad(wel1, ((0, 0), (0, ED - H)))], axis=1)

    few0t = We0.reshape(H, F0).T
    few1t = We1.reshape(H, ED).T
    ae0t = ae0.T
    ae1t = ae1.T
    b0t = b0.reshape(H, F0).T
    b1t = b1.reshape(H, ED).T

    WlfT = jnp.pad(Wl.reshape(N, ED, ED).transpose(1, 0, 2),
                   ((0, 0), (0, NP - N), (0, 0))).reshape(K2, ED)
    blr = bl.reshape(1, ED)

    rep2 = lambda i: (0, 0)
    h1t = pl.pallas_call(
        _gnn_block,
        grid=(BS // BB,),
        in_specs=[
            pl.BlockSpec((BB, 8, NP), lambda i: (i, 0, 0)),
            pl.BlockSpec((BB, J, N), lambda i: (i, 0, 0)),
            pl.BlockSpec((BB, J, J), lambda i: (i, 0, 0)),
            pl.BlockSpec((8, F0 * 4), rep2),
            pl.BlockSpec((8, 1), rep2),
            pl.BlockSpec((8, 1), rep2),
            pl.BlockSpec((8, H), rep2),
            pl.BlockSpec((F0, H), rep2),
            pl.BlockSpec((F0, H), rep2),
            pl.BlockSpec((F0, H), rep2),
            pl.BlockSpec((F0, H * ED + ED), rep2),
            pl.BlockSpec((F0, H), rep2),
            pl.BlockSpec((ED, H), rep2),
            pl.BlockSpec((ED, H), rep2),
            pl.BlockSpec((ED, H), rep2),
        ],
        out_specs=pl.BlockSpec((BB, ED, NP), lambda i: (i, 0, 0)),
        out_shape=jax.ShapeDtypeStruct((BS, ED, NP), jnp.float32),
    )(XT, G3, T, W0aug, g8, b8, wer0, few0t, ae0t, b0t,
      W1aug, wer1, few1t, ae1t, b1t)

    out = pl.pallas_call(
        _fin_block,
        grid=(NK,),
        in_specs=[
            pl.BlockSpec((BS, KB), lambda k: (0, k)),
            pl.BlockSpec((KB, ED), lambda k: (k, 0)),
            pl.BlockSpec((1, ED), lambda k: (0, 0)),
        ],
        out_specs=pl.BlockSpec((BS, ED), lambda k: (0, 0)),
        out_shape=jax.ShapeDtypeStruct((BS, ED), jnp.float32),
    )(h1t.reshape(BS, K2), WlfT, blr)
    return out


# speedup vs baseline: 7.5443x; 1.1523x over previous
import jax
import jax.numpy as jnp
from jax.experimental import pallas as pl

J = 100
M = 20
N = J + M
NP = 128
BS = 256
H = 3
F0 = 16
ED = 128
BB = 16
FB = 16
NFB = ED // FB


def _lrelu(x, s):
    return jnp.maximum(x, s * x)


def _mm(x3, w):
    b, n, k = x3.shape
    y = jnp.dot(x3.reshape(b * n, k), w, preferred_element_type=jnp.float32)
    return y.reshape(b, n, -1)


def _gnn_block(nh_ref, nl_ref, nw_ref, np_ref, nn_ref, a_ref, t_ref,
               w0_ref, g_ref, bln_ref,
               wer0_ref, few0t_ref, ae0t_ref, b0t_ref,
               w1_ref, wer1_ref, few1t_ref, ae1t_ref, b1t_ref,
               h1_ref):
    zlane = jnp.zeros((BB, NP - J), jnp.float32)
    zrow = jnp.zeros((BB, NP), jnp.float32)
    lane = jax.lax.broadcasted_iota(jnp.int32, (BB, NP), 1)
    mJ = lane < J
    rows = [jnp.concatenate([nh_ref[...], zlane], axis=1),
            jnp.concatenate([nl_ref[...], zlane], axis=1),
            jnp.where(mJ, nw_ref[...], 0.0),
            jnp.where(mJ, np_ref[...], 0.0),
            jnp.where(mJ, nn_ref[...], 0.0),
            zrow, zrow, zrow]
    XT = jnp.stack(rows, axis=1)

    row = jax.lax.broadcasted_iota(jnp.int32, XT.shape, 1)
    m5 = (row < 5).astype(jnp.float32)
    mu = jnp.sum(XT, axis=1, keepdims=True) * (1.0 / 5.0)
    d = (XT - mu) * m5
    var = jnp.sum(d * d, axis=1, keepdims=True) * (1.0 / 5.0)
    XnT = d * jax.lax.rsqrt(var + 1e-5) * g_ref[...].reshape(1, 8, 1) \
        + bln_ref[...].reshape(1, 8, 1)

    G = a_ref[...]
    Gp = jnp.concatenate([G, jnp.zeros((BB, NP - J, N), jnp.float32)], axis=1)
    Gp = jnp.concatenate([Gp, jnp.zeros((BB, NP, NP - N), jnp.float32)], axis=2)
    Ab = Gp > 0
    Tr = t_ref[...]
    Tm = jnp.concatenate([Tr, jnp.zeros((BB, NP - J, J), jnp.float32)], axis=1)
    Tm = jnp.concatenate([Tm, jnp.zeros((BB, NP, NP - J), jnp.float32)], axis=2)

    def gat_T(ftaug, srcT, elbase, wer_ref, fewt_ref, aet_ref, bt_ref, Fh):
        C = srcT.shape[1]
        acc = None
        for h in range(H):
            fth = ftaug[:, :, h * Fh:(h + 1) * Fh]
            el = ftaug[:, :, elbase + h:elbase + h + 1]
            er = jnp.sum(srcT * wer_ref[:, h:h + 1].reshape(1, C, 1),
                         axis=1, keepdims=True)
            eec = jnp.sum(fewt_ref[:, h:h + 1] * aet_ref[:, h:h + 1])
            logits = el + er + Tm * eec
            logits = _lrelu(logits, 0.2)
            logits = jnp.where(Ab, logits, -1e9)
            mx = jnp.max(logits, axis=1, keepdims=True)
            ex = jnp.where(Ab, jnp.exp(logits - mx), 0.0)
            den = jnp.sum(ex, axis=1, keepdims=True)
            rden = jnp.where(den > 0, 1.0 / den, 0.0)
            outT = jax.lax.dot_general(
                fth, ex, (((1,), (1,)), ((0,), (0,))),
                preferred_element_type=jnp.float32)
            eaggT = jnp.sum(ex * Tm, axis=1, keepdims=True)
            fewcol = fewt_ref[:, h:h + 1].reshape(1, Fh, 1)
            bcol = bt_ref[:, h:h + 1].reshape(1, Fh, 1)
            hh = _lrelu((outT + eaggT * fewcol) * rden + bcol, 0.01)
            acc = hh if acc is None else acc + hh
        return acc * (1.0 / H)

    Xn = jnp.swapaxes(XnT, 1, 2)
    ft0aug = _mm(Xn, w0_ref[...])
    h0T = gat_T(ft0aug, XnT, 3 * F0, wer0_ref, few0t_ref, ae0t_ref,
                b0t_ref, F0)
    h0 = jnp.swapaxes(h0T, 1, 2)
    ft1aug = _mm(h0, w1_ref[...])
    h1_ref[...] = gat_T(ft1aug, h0T, H * ED, wer1_ref, few1t_ref, ae1t_ref,
                        b1t_ref, ED)


def _fin_block(h_ref, w_ref, b_ref, o_ref):
    k = pl.program_id(0)
    hblk = h_ref[...]
    wblk = jnp.concatenate(
        [w_ref[...], jnp.zeros((NP - N, FB, ED), jnp.float32)], axis=0)
    part = None
    for ff in range(FB):
        p = jnp.dot(hblk[:, ff, :], wblk[:, ff, :],
                    preferred_element_type=jnp.float32)
        part = p if part is None else part + p

    @pl.when(k == 0)
    def _():
        o_ref[...] = part

    @pl.when(k > 0)
    def _():
        o_ref[...] += part

    @pl.when(k == NFB - 1)
    def _():
        o_ref[...] = _lrelu(o_ref[...] + b_ref[...], 0.01)


def kernel(Graph, norm_h, norm_L, norm_W, norm_P, norm_N, T, ln_g, ln_b,
           W0, We0, al0, ar0, ae0, b0, W1, We1, al1, ar1, ae1, b1, Wl, bl):
    G3 = Graph.reshape(BS, J, N)

    g8 = jnp.pad(ln_g, (0, 3)).reshape(8, 1)
    b8 = jnp.pad(ln_b, (0, 3)).reshape(8, 1)

    W0r = W0.reshape(5, H, F0)
    wel0 = jnp.einsum('chf,hf->ch', W0r, al0)
    wer0 = jnp.pad(jnp.einsum('chf,hf->ch', W0r, ar0), ((0, 3), (0, 0)))
    W0aug = jnp.concatenate(
        [jnp.pad(W0, ((0, 3), (0, 0))),
         jnp.pad(wel0, ((0, 3), (0, F0 - H)))], axis=1)
    W1r = W1.reshape(F0, H, ED)
    wel1 = jnp.einsum('chf,hf->ch', W1r, al1)
    wer1 = jnp.einsum('chf,hf->ch', W1r, ar1)
    W1aug = jnp.concatenate(
        [W1, jnp.pad(wel1, ((0, 0), (0, ED - H)))], axis=1)

    few0t = We0.reshape(H, F0).T
    few1t = We1.reshape(H, ED).T
    ae0t = ae0.T
    ae1t = ae1.T
    b0t = b0.reshape(H, F0).T
    b1t = b1.reshape(H, ED).T

    Wl3 = Wl.reshape(N, ED, ED)
    blr = bl.reshape(1, ED)

    rep2 = lambda i: (0, 0)
    h1t = pl.pallas_call(
        _gnn_block,
        grid=(BS // BB,),
        in_specs=[
            pl.BlockSpec((BB, J), lambda i: (i, 0)),
            pl.BlockSpec((BB, J), lambda i: (i, 0)),
            pl.BlockSpec((BB, 1), lambda i: (i, 0)),
            pl.BlockSpec((BB, 1), lambda i: (i, 0)),
            pl.BlockSpec((BB, 1), lambda i: (i, 0)),
            pl.BlockSpec((BB, J, N), lambda i: (i, 0, 0)),
            pl.BlockSpec((BB, J, J), lambda i: (i, 0, 0)),
            pl.BlockSpec((8, F0 * 4), rep2),
            pl.BlockSpec((8, 1), rep2),
            pl.BlockSpec((8, 1), rep2),
            pl.BlockSpec((8, H), rep2),
            pl.BlockSpec((F0, H), rep2),
            pl.BlockSpec((F0, H), rep2),
            pl.BlockSpec((F0, H), rep2),
            pl.BlockSpec((F0, H * ED + ED), rep2),
            pl.BlockSpec((F0, H), rep2),
            pl.BlockSpec((ED, H), rep2),
            pl.BlockSpec((ED, H), rep2),
            pl.BlockSpec((ED, H), rep2),
        ],
        out_specs=pl.BlockSpec((BB, ED, NP), lambda i: (i, 0, 0)),
        out_shape=jax.ShapeDtypeStruct((BS, ED, NP), jnp.float32),
    )(norm_h, norm_L, norm_W, norm_P, norm_N, G3, T, W0aug, g8, b8,
      wer0, few0t, ae0t, b0t, W1aug, wer1, few1t, ae1t, b1t)

    out = pl.pallas_call(
        _fin_block,
        grid=(NFB,),
        in_specs=[
            pl.BlockSpec((BS, FB, NP), lambda k: (0, k, 0)),
            pl.BlockSpec((N, FB, ED), lambda k: (0, k, 0)),
            pl.BlockSpec((1, ED), lambda k: (0, 0)),
        ],
        out_specs=pl.BlockSpec((BS, ED), lambda k: (0, 0)),
        out_shape=jax.ShapeDtypeStruct((BS, ED), jnp.float32),
    )(h1t, Wl3, blr)
    return out


# speedup vs baseline: 7.7467x; 1.0268x over previous
import jax
import jax.numpy as jnp
from jax.experimental import pallas as pl
from jax.experimental.pallas import tpu as pltpu

J = 100
M = 20
N = J + M
NP = 128
BS = 256
H = 3
F0 = 16
ED = 128
BB = 16
FB = 16
NFB = ED // FB


def _lrelu(x, s):
    return jnp.maximum(x, s * x)


def _mm(x3, w):
    b, n, k = x3.shape
    y = jnp.dot(x3.reshape(b * n, k), w, preferred_element_type=jnp.float32)
    return y.reshape(b, n, -1)


def _gnn_block(nh_ref, nl_ref, nw_ref, np_ref, nn_ref, a_ref, t_ref,
               g_ref, bln_ref, w0_ref, al0_ref, ar0_ref, ae0_ref, we0_ref,
               b0_ref, w1_ref, al1_ref, ar1_ref, ae1_ref, we1_ref, b1_ref,
               h1_ref):
    zlane = jnp.zeros((BB, NP - J), jnp.float32)
    lane = jax.lax.broadcasted_iota(jnp.int32, (BB, NP), 1)
    mJ = lane < J
    rows = [jnp.concatenate([nh_ref[...], zlane], axis=1),
            jnp.concatenate([nl_ref[...], zlane], axis=1),
            jnp.where(mJ, nw_ref[...], 0.0),
            jnp.where(mJ, np_ref[...], 0.0),
            jnp.where(mJ, nn_ref[...], 0.0)]
    XT = jnp.stack(rows, axis=1)

    mu = jnp.sum(XT, axis=1, keepdims=True) * (1.0 / 5.0)
    d = XT - mu
    var = jnp.sum(d * d, axis=1, keepdims=True) * (1.0 / 5.0)
    zn = d * jax.lax.rsqrt(var + 1e-5)
    XnT = jnp.concatenate(
        [zn[:, c:c + 1, :] * g_ref[c] + bln_ref[c] for c in range(5)]
        + [jnp.zeros((BB, 3, NP), jnp.float32)], axis=1)

    G = a_ref[...]
    Gp = jnp.concatenate([G, jnp.zeros((BB, NP - J, N), jnp.float32)], axis=1)
    Gp = jnp.concatenate([Gp, jnp.zeros((BB, NP, NP - N), jnp.float32)], axis=2)
    Ab = Gp > 0
    Tr = t_ref[...]
    Tm = jnp.concatenate([Tr, jnp.zeros((BB, NP - J, J), jnp.float32)], axis=1)
    Tm = jnp.concatenate([Tm, jnp.zeros((BB, NP, NP - J), jnp.float32)], axis=2)

    def prep(w_ref, al_ref, ar_ref, Fh, Cpad):
        wel, wer = [], []
        for h in range(H):
            wsl = w_ref[:, h * Fh:(h + 1) * Fh]
            wel.append(jnp.sum(wsl * al_ref[h:h + 1, :], axis=1, keepdims=True))
            wer.append(jnp.sum(wsl * ar_ref[h:h + 1, :], axis=1, keepdims=True))
        C = wel[0].shape[0]
        welm = jnp.concatenate(wel, axis=1)
        if Cpad > C:
            welm = jnp.concatenate(
                [welm, jnp.zeros((Cpad - C, H), jnp.float32)], axis=0)
        wercols = [jnp.concatenate(
            [w, jnp.zeros((Cpad - C, 1), jnp.float32)], axis=0).reshape(1, Cpad, 1)
            if Cpad > C else w.reshape(1, Cpad, 1) for w in wer]
        return welm, wercols

    def gat_T(ft, el3, srcT, wercols, we_ref, ae_ref, b_ref, Fh):
        acc = None
        for h in range(H):
            fth = ft[:, :, h * Fh:(h + 1) * Fh]
            el = el3[:, :, h:h + 1]
            er = jnp.sum(srcT * wercols[h], axis=1, keepdims=True)
            wesl = we_ref[:, h * Fh:(h + 1) * Fh]
            eec = jnp.sum(wesl * ae_ref[h:h + 1, :])
            logits = el + er + Tm * eec
            logits = _lrelu(logits, 0.2)
            logits = jnp.where(Ab, logits, -1e9)
            mx = jnp.max(logits, axis=1, keepdims=True)
            ex = jnp.exp(logits - mx)
            den = jnp.sum(ex, axis=1, keepdims=True)
            rden = jnp.where(mx > -1e8, 1.0 / den, 0.0)
            outT = jax.lax.dot_general(
                fth, ex, (((1,), (1,)), ((0,), (0,))),
                preferred_element_type=jnp.float32)
            eaggT = jnp.sum(ex * Tm, axis=1, keepdims=True)
            fewcol = jnp.swapaxes(wesl, 0, 1).reshape(1, Fh, 1)
            bcol = jnp.swapaxes(b_ref[:, h * Fh:(h + 1) * Fh],
                                0, 1).reshape(1, Fh, 1)
            hh = _lrelu((outT + eaggT * fewcol) * rden + bcol, 0.01)
            acc = hh if acc is None else acc + hh
        return acc * (1.0 / H)

    Xn = jnp.swapaxes(XnT, 1, 2)
    W0p = jnp.concatenate(
        [w0_ref[...], jnp.zeros((3, H * F0), jnp.float32)], axis=0)
    wel0, wer0 = prep(w0_ref, al0_ref, ar0_ref, F0, 8)
    ft0 = _mm(Xn, W0p)
    el03 = _mm(Xn, wel0)
    h0T = gat_T(ft0, el03, XnT, wer0, we0_ref, ae0_ref, b0_ref, F0)

    h0 = jnp.swapaxes(h0T, 1, 2)
    wel1, wer1 = prep(w1_ref, al1_ref, ar1_ref, ED, F0)
    ft1 = _mm(h0, w1_ref[...])
    el13 = _mm(h0, wel1)
    h1_ref[...] = gat_T(ft1, el13, h0T, wer1, we1_ref, ae1_ref, b1_ref, ED)


def _fin_block(h_ref, w_ref, b_ref, o_ref):
    k = pl.program_id(0)
    hblk = h_ref[...]
    wblk = w_ref[...]
    part = None
    for ff in range(FB):
        p = jnp.dot(hblk[:, ff, :N], wblk[:, ff, :],
                    preferred_element_type=jnp.float32)
        part = p if part is None else part + p

    @pl.when(k == 0)
    def _():
        o_ref[...] = part

    @pl.when(k > 0)
    def _():
        o_ref[...] += part

    @pl.when(k == NFB - 1)
    def _():
        o_ref[...] = _lrelu(o_ref[...] + b_ref[...], 0.01)


def kernel(Graph, norm_h, norm_L, norm_W, norm_P, norm_N, T, ln_g, ln_b,
           W0, We0, al0, ar0, ae0, b0, W1, We1, al1, ar1, ae1, b1, Wl, bl):
    G3 = Graph.reshape(BS, J, N)
    b0r = b0.reshape(1, H * F0)
    b1r = b1.reshape(1, H * ED)
    Wl3 = Wl.reshape(N, ED, ED)
    blr = bl.reshape(1, ED)

    rep2 = lambda i: (0, 0)
    smem = pl.BlockSpec(memory_space=pltpu.SMEM)
    h1t = pl.pallas_call(
        _gnn_block,
        grid=(BS // BB,),
        in_specs=[
            pl.BlockSpec((BB, J), lambda i: (i, 0)),
            pl.BlockSpec((BB, J), lambda i: (i, 0)),
            pl.BlockSpec((BB, 1), lambda i: (i, 0)),
            pl.BlockSpec((BB, 1), lambda i: (i, 0)),
            pl.BlockSpec((BB, 1), lambda i: (i, 0)),
            pl.BlockSpec((BB, J, N), lambda i: (i, 0, 0)),
            pl.BlockSpec((BB, J, J), lambda i: (i, 0, 0)),
            smem,
            smem,
            pl.BlockSpec((5, H * F0), rep2),
            pl.BlockSpec((H, F0), rep2),
            pl.BlockSpec((H, F0), rep2),
            pl.BlockSpec((H, F0), rep2),
            pl.BlockSpec((1, H * F0), rep2),
            pl.BlockSpec((1, H * F0), rep2),
            pl.BlockSpec((F0, H * ED), rep2),
            pl.BlockSpec((H, ED), rep2),
            pl.BlockSpec((H, ED), rep2),
            pl.BlockSpec((H, ED), rep2),
            pl.BlockSpec((1, H * ED), rep2),
            pl.BlockSpec((1, H * ED), rep2),
        ],
        out_specs=pl.BlockSpec((BB, ED, NP), lambda i: (i, 0, 0)),
        out_shape=jax.ShapeDtypeStruct((BS, ED, NP), jnp.float32),
    )(norm_h, norm_L, norm_W, norm_P, norm_N, G3, T, ln_g, ln_b,
      W0, al0, ar0, ae0, We0, b0r, W1, al1, ar1, ae1, We1, b1r)

    out = pl.pallas_call(
        _fin_block,
        grid=(NFB,),
        in_specs=[
            pl.BlockSpec((BS, FB, NP), lambda k: (0, k, 0)),
            pl.BlockSpec((N, FB, ED), lambda k: (0, k, 0)),
            pl.BlockSpec((1, ED), lambda k: (0, 0)),
        ],
        out_specs=pl.BlockSpec((BS, ED), lambda k: (0, 0)),
        out_shape=jax.ShapeDtypeStruct((BS, ED), jnp.float32),
    )(h1t, Wl3, blr)
    return out


# speedup vs baseline: 9.2867x; 1.1988x over previous
import jax
import jax.numpy as jnp
from jax.experimental import pallas as pl
from jax.experimental.pallas import tpu as pltpu

J = 100
M = 20
N = J + M
NP = 128
BS = 256
H = 3
F0 = 16
ED = 128
BB = 16
FB = 16
NFB = ED // FB


def _lrelu(x, s):
    return jnp.maximum(x, s * x)


def _mm(x3, w):
    b, n, k = x3.shape
    y = jnp.dot(x3.reshape(b * n, k), w, preferred_element_type=jnp.float32)
    return y.reshape(b, n, -1)


def _gnn_block(nh_ref, nl_ref, nw_ref, np_ref, nn_ref, a_ref, t_ref,
               g_ref, bln_ref, w0_ref, al0_ref, ar0_ref, ae0_ref, we0_ref,
               b0_ref, w1_ref, al1_ref, ar1_ref, ae1_ref, we1_ref, b1_ref,
               h1_ref,
               w0p_s, wel0_s, wer0_s, few0_s, b0c_s,
               wel1_s, wer1_s, few1_s, b1c_s, eec_s):
    @pl.when(pl.program_id(0) == 0)
    def _():
        w0p_s[...] = jnp.concatenate(
            [w0_ref[...], jnp.zeros((3, H * F0), jnp.float32)], axis=0)
        for (w_ref, al_ref, ar_ref, ae_ref, we_ref, b_ref, Fh, Cpad, wel_s,
             wer_s, few_s, bc_s, ebase) in (
                (w0_ref, al0_ref, ar0_ref, ae0_ref, we0_ref, b0_ref, F0, 8,
                 wel0_s, wer0_s, few0_s, b0c_s, 0),
                (w1_ref, al1_ref, ar1_ref, ae1_ref, we1_ref, b1_ref, ED, F0,
                 wel1_s, wer1_s, few1_s, b1c_s, H)):
            C = w_ref.shape[0]
            zpad = jnp.zeros((max(Cpad - C, 1), 1), jnp.float32)
            for h in range(H):
                wsl = w_ref[:, h * Fh:(h + 1) * Fh]
                wel = jnp.sum(wsl * al_ref[h:h + 1, :], axis=1, keepdims=True)
                wer = jnp.sum(wsl * ar_ref[h:h + 1, :], axis=1, keepdims=True)
                wesl = we_ref[:, h * Fh:(h + 1) * Fh]
                if Cpad > C:
                    wel = jnp.concatenate([wel, zpad], axis=0)
                    wer = jnp.concatenate([wer, zpad], axis=0)
                wel_s[:, h:h + 1] = wel
                wer_s[:, h:h + 1] = wer
                few_s[:, h:h + 1] = jnp.swapaxes(wesl, 0, 1)
                bc_s[:, h:h + 1] = jnp.swapaxes(
                    b_ref[:, h * Fh:(h + 1) * Fh], 0, 1)
                eec_s[ebase + h] = jnp.sum(wesl * ae_ref[h:h + 1, :])

    zlane = jnp.zeros((BB, NP - J), jnp.float32)
    lane = jax.lax.broadcasted_iota(jnp.int32, (BB, NP), 1)
    mJ = lane < J
    rows = [jnp.concatenate([nh_ref[...], zlane], axis=1),
            jnp.concatenate([nl_ref[...], zlane], axis=1),
            jnp.where(mJ, nw_ref[...], 0.0),
            jnp.where(mJ, np_ref[...], 0.0),
            jnp.where(mJ, nn_ref[...], 0.0)]
    XT = jnp.stack(rows, axis=1)

    mu = jnp.sum(XT, axis=1, keepdims=True) * (1.0 / 5.0)
    d = XT - mu
    var = jnp.sum(d * d, axis=1, keepdims=True) * (1.0 / 5.0)
    zn = d * jax.lax.rsqrt(var + 1e-5)
    XnT = jnp.concatenate(
        [zn[:, c:c + 1, :] * g_ref[c] + bln_ref[c] for c in range(5)]
        + [jnp.zeros((BB, 3, NP), jnp.float32)], axis=1)

    G = a_ref[...]
    Gp = jnp.concatenate([G, jnp.zeros((BB, NP - J, N), jnp.float32)], axis=1)
    Gp = jnp.concatenate([Gp, jnp.zeros((BB, NP, NP - N), jnp.float32)], axis=2)
    Ab = Gp > 0
    Tr = t_ref[...]
    Tm = jnp.concatenate([Tr, jnp.zeros((BB, NP - J, J), jnp.float32)], axis=1)
    Tm = jnp.concatenate([Tm, jnp.zeros((BB, NP, NP - J), jnp.float32)], axis=2)

    def gat_T(ft, el3, srcT, wer_s, few_s, bc_s, Fh, ebase):
        C = srcT.shape[1]
        acc = None
        for h in range(H):
            fth = ft[:, :, h * Fh:(h + 1) * Fh]
            el = el3[:, :, h:h + 1]
            er = jnp.sum(srcT * wer_s[:, h:h + 1].reshape(1, C, 1),
                         axis=1, keepdims=True)
            eec = eec_s[ebase + h]
            logits = el + er + Tm * eec
            logits = _lrelu(logits, 0.2)
            logits = jnp.where(Ab, logits, -1e9)
            mx = jnp.max(logits, axis=1, keepdims=True)
            ex = jnp.exp(logits - mx)
            den = jnp.sum(ex, axis=1, keepdims=True)
            rden = jnp.where(mx > -1e8, 1.0 / den, 0.0)
            outT = jax.lax.dot_general(
                fth, ex, (((1,), (1,)), ((0,), (0,))),
                preferred_element_type=jnp.float32)
            eaggT = jnp.sum(ex * Tm, axis=1, keepdims=True)
            fewcol = few_s[:, h:h + 1].reshape(1, Fh, 1)
            bcol = bc_s[:, h:h + 1].reshape(1, Fh, 1)
            hh = _lrelu((outT + eaggT * fewcol) * rden + bcol, 0.01)
            acc = hh if acc is None else acc + hh
        return acc * (1.0 / H)

    Xn = jnp.swapaxes(XnT, 1, 2)
    ft0 = _mm(Xn, w0p_s[...])
    el03 = _mm(Xn, wel0_s[...])
    h0T = gat_T(ft0, el03, XnT, wer0_s, few0_s, b0c_s, F0, 0)

    h0 = jnp.swapaxes(h0T, 1, 2)
    ft1 = _mm(h0, w1_ref[...])
    el13 = _mm(h0, wel1_s[...])
    h1_ref[...] = gat_T(ft1, el13, h0T, wer1_s, few1_s, b1c_s, ED, H)


def _fin_block(h_ref, w_ref, b_ref, o_ref):
    k = pl.program_id(0)
    hblk = h_ref[...]
    wblk = w_ref[...]
    part = None
    for ff in range(FB):
        p = jnp.dot(hblk[:, ff, :N], wblk[:, ff, :],
                    preferred_element_type=jnp.float32)
        part = p if part is None else part + p

    @pl.when(k == 0)
    def _():
        o_ref[...] = part

    @pl.when(k > 0)
    def _():
        o_ref[...] += part

    @pl.when(k == NFB - 1)
    def _():
        o_ref[...] = _lrelu(o_ref[...] + b_ref[...], 0.01)


def kernel(Graph, norm_h, norm_L, norm_W, norm_P, norm_N, T, ln_g, ln_b,
           W0, We0, al0, ar0, ae0, b0, W1, We1, al1, ar1, ae1, b1, Wl, bl):
    G3 = Graph.reshape(BS, J, N)
    b0r = b0.reshape(1, H * F0)
    b1r = b1.reshape(1, H * ED)
    Wl3 = Wl.reshape(N, ED, ED)
    blr = bl.reshape(1, ED)

    rep2 = lambda i: (0, 0)
    smem = pl.BlockSpec(memory_space=pltpu.SMEM)
    h1t = pl.pallas_call(
        _gnn_block,
        grid=(BS // BB,),
        in_specs=[
            pl.BlockSpec((BB, J), lambda i: (i, 0)),
            pl.BlockSpec((BB, J), lambda i: (i, 0)),
            pl.BlockSpec((BB, 1), lambda i: (i, 0)),
            pl.BlockSpec((BB, 1), lambda i: (i, 0)),
            pl.BlockSpec((BB, 1), lambda i: (i, 0)),
            pl.BlockSpec((BB, J, N), lambda i: (i, 0, 0)),
            pl.BlockSpec((BB, J, J), lambda i: (i, 0, 0)),
            smem,
            smem,
            pl.BlockSpec((5, H * F0), rep2),
            pl.BlockSpec((H, F0), rep2),
            pl.BlockSpec((H, F0), rep2),
            pl.BlockSpec((H, F0), rep2),
            pl.BlockSpec((1, H * F0), rep2),
            pl.BlockSpec((1, H * F0), rep2),
            pl.BlockSpec((F0, H * ED), rep2),
            pl.BlockSpec((H, ED), rep2),
            pl.BlockSpec((H, ED), rep2),
            pl.BlockSpec((H, ED), rep2),
            pl.BlockSpec((1, H * ED), rep2),
            pl.BlockSpec((1, H * ED), rep2),
        ],
        out_specs=pl.BlockSpec((BB, ED, NP), lambda i: (i, 0, 0)),
        out_shape=jax.ShapeDtypeStruct((BS, ED, NP), jnp.float32),
        scratch_shapes=[
            pltpu.VMEM((8, H * F0), jnp.float32),
            pltpu.VMEM((8, H), jnp.float32),
            pltpu.VMEM((8, H), jnp.float32),
            pltpu.VMEM((F0, H), jnp.float32),
            pltpu.VMEM((F0, H), jnp.float32),
            pltpu.VMEM((F0, H), jnp.float32),
            pltpu.VMEM((F0, H), jnp.float32),
            pltpu.VMEM((ED, H), jnp.float32),
            pltpu.VMEM((ED, H), jnp.float32),
            pltpu.SMEM((2 * H,), jnp.float32),
        ],
    )(norm_h, norm_L, norm_W, norm_P, norm_N, G3, T, ln_g, ln_b,
      W0, al0, ar0, ae0, We0, b0r, W1, al1, ar1, ae1, We1, b1r)

    out = pl.pallas_call(
        _fin_block,
        grid=(NFB,),
        in_specs=[
            pl.BlockSpec((BS, FB, NP), lambda k: (0, k, 0)),
            pl.BlockSpec((N, FB, ED), lambda k: (0, k, 0)),
            pl.BlockSpec((1, ED), lambda k: (0, 0)),
        ],
        out_specs=pl.BlockSpec((BS, ED), lambda k: (0, 0)),
        out_shape=jax.ShapeDtypeStruct((BS, ED), jnp.float32),
    )(h1t, Wl3, blr)
    return out


# speedup vs baseline: 9.4269x; 1.0151x over previous
import jax
import jax.numpy as jnp
from jax.experimental import pallas as pl
from jax.experimental.pallas import tpu as pltpu

J = 100
M = 20
N = J + M
NP = 128
SP = 104
BS = 256
H = 3
F0 = 16
ED = 128
BB = 16
FB = 16
NFB = ED // FB


def _lrelu(x, s):
    return jnp.maximum(x, s * x)


def _mm(x3, w):
    b, n, k = x3.shape
    y = jnp.dot(x3.reshape(b * n, k), w, preferred_element_type=jnp.float32)
    return y.reshape(b, n, -1)


def _gnn_block(nh_ref, nl_ref, nw_ref, np_ref, nn_ref, a_ref, t_ref,
               g_ref, bln_ref, w0_ref, al0_ref, ar0_ref, ae0_ref, we0_ref,
               b0_ref, w1_ref, al1_ref, ar1_ref, ae1_ref, we1_ref, b1_ref,
               h1_ref,
               w0p_s, wel0_s, wer0_s, few0_s, b0c_s,
               wel1_s, wer1_s, few1_s, b1c_s, eec_s):
    @pl.when(pl.program_id(0) == 0)
    def _():
        w0p_s[...] = jnp.concatenate(
            [w0_ref[...], jnp.zeros((3, H * F0), jnp.float32)], axis=0)
        for (w_ref, al_ref, ar_ref, ae_ref, we_ref, b_ref, Fh, Cpad, wel_s,
             wer_s, few_s, bc_s, ebase) in (
                (w0_ref, al0_ref, ar0_ref, ae0_ref, we0_ref, b0_ref, F0, 8,
                 wel0_s, wer0_s, few0_s, b0c_s, 0),
                (w1_ref, al1_ref, ar1_ref, ae1_ref, we1_ref, b1_ref, ED, F0,
                 wel1_s, wer1_s, few1_s, b1c_s, H)):
            C = w_ref.shape[0]
            zpad = jnp.zeros((max(Cpad - C, 1), 1), jnp.float32)
            for h in range(H):
                wsl = w_ref[:, h * Fh:(h + 1) * Fh]
                wel = jnp.sum(wsl * al_ref[h:h + 1, :], axis=1, keepdims=True)
                wer = jnp.sum(wsl * ar_ref[h:h + 1, :], axis=1, keepdims=True)
                wesl = we_ref[:, h * Fh:(h + 1) * Fh]
                if Cpad > C:
                    wel = jnp.concatenate([wel, zpad], axis=0)
                    wer = jnp.concatenate([wer, zpad], axis=0)
                wel_s[:, h:h + 1] = wel
                wer_s[:, h:h + 1] = wer
                few_s[:, h:h + 1] = jnp.swapaxes(wesl, 0, 1)
                bc_s[:, h:h + 1] = jnp.swapaxes(
                    b_ref[:, h * Fh:(h + 1) * Fh], 0, 1)
                eec_s[ebase + h] = jnp.sum(wesl * ae_ref[h:h + 1, :])

    zlane = jnp.zeros((BB, NP - J), jnp.float32)
    lane = jax.lax.broadcasted_iota(jnp.int32, (BB, NP), 1)
    mJ = lane < J
    rows = [jnp.concatenate([nh_ref[...], zlane], axis=1),
            jnp.concatenate([nl_ref[...], zlane], axis=1),
            jnp.where(mJ, nw_ref[...], 0.0),
            jnp.where(mJ, np_ref[...], 0.0),
            jnp.where(mJ, nn_ref[...], 0.0)]
    XT = jnp.stack(rows, axis=1)

    mu = jnp.sum(XT, axis=1, keepdims=True) * (1.0 / 5.0)
    d = XT - mu
    var = jnp.sum(d * d, axis=1, keepdims=True) * (1.0 / 5.0)
    zn = d * jax.lax.rsqrt(var + 1e-5)
    XnT = jnp.concatenate(
        [zn[:, c:c + 1, :] * g_ref[c] + bln_ref[c] for c in range(5)]
        + [jnp.zeros((BB, 3, NP), jnp.float32)], axis=1)

    G = a_ref[...]
    Gp = jnp.concatenate([G, jnp.zeros((BB, SP - J, N), jnp.float32)], axis=1)
    Gp = jnp.concatenate([Gp, jnp.zeros((BB, SP, NP - N), jnp.float32)], axis=2)
    Ab = Gp > 0
    Tr = t_ref[...]
    Tm = jnp.concatenate([Tr, jnp.zeros((BB, SP - J, J), jnp.float32)], axis=1)
    Tm = jnp.concatenate([Tm, jnp.zeros((BB, SP, NP - J), jnp.float32)], axis=2)

    def gat_T(ft, el3, srcT, wer_s, few_s, bc_s, Fh, ebase):
        C = srcT.shape[1]
        acc = None
        for h in range(H):
            fth = ft[:, :, h * Fh:(h + 1) * Fh]
            el = el3[:, :, h:h + 1]
            er = jnp.sum(srcT * wer_s[:, h:h + 1].reshape(1, C, 1),
                         axis=1, keepdims=True)
            eec = eec_s[ebase + h]
            logits = el + er + Tm * eec
            logits = _lrelu(logits, 0.2)
            logits = jnp.where(Ab, logits, -1e9)
            mx = jnp.max(logits, axis=1, keepdims=True)
            ex = jnp.exp(logits - mx)
            den = jnp.sum(ex, axis=1, keepdims=True)
            rden = jnp.where(mx > -1e8, 1.0 / den, 0.0)
            outT = jax.lax.dot_general(
                fth, ex, (((1,), (1,)), ((0,), (0,))),
                preferred_element_type=jnp.float32)
            eaggT = jnp.sum(ex * Tm, axis=1, keepdims=True)
            fewcol = few_s[:, h:h + 1].reshape(1, Fh, 1)
            bcol = bc_s[:, h:h + 1].reshape(1, Fh, 1)
            hh = _lrelu((outT + eaggT * fewcol) * rden + bcol, 0.01)
            acc = hh if acc is None else acc + hh
        return acc * (1.0 / H)

    Xn = jnp.swapaxes(XnT[:, :, :SP], 1, 2)
    ft0 = _mm(Xn, w0p_s[...])
    el03 = _mm(Xn, wel0_s[...])
    h0T = gat_T(ft0, el03, XnT, wer0_s, few0_s, b0c_s, F0, 0)

    h0 = jnp.swapaxes(h0T[:, :, :SP], 1, 2)
    ft1 = _mm(h0, w1_ref[...])
    el13 = _mm(h0, wel1_s[...])
    h1_ref[...] = gat_T(ft1, el13, h0T, wer1_s, few1_s, b1c_s, ED, H)


def _fin_block(h_ref, w_ref, b_ref, o_ref):
    k = pl.program_id(0)
    hblk = h_ref[...]
    wblk = w_ref[...]
    part = None
    for ff in range(FB):
        p = jnp.dot(hblk[:, ff, :N], wblk[:, ff, :],
                    preferred_element_type=jnp.float32)
        part = p if part is None else part + p

    @pl.when(k == 0)
    def _():
        o_ref[...] = part

    @pl.when(k > 0)
    def _():
        o_ref[...] += part

    @pl.when(k == NFB - 1)
    def _():
        o_ref[...] = _lrelu(o_ref[...] + b_ref[...], 0.01)


def kernel(Graph, norm_h, norm_L, norm_W, norm_P, norm_N, T, ln_g, ln_b,
           W0, We0, al0, ar0, ae0, b0, W1, We1, al1, ar1, ae1, b1, Wl, bl):
    G3 = Graph.reshape(BS, J, N)
    b0r = b0.reshape(1, H * F0)
    b1r = b1.reshape(1, H * ED)
    Wl3 = Wl.reshape(N, ED, ED)
    blr = bl.reshape(1, ED)

    rep2 = lambda i: (0, 0)
    smem = pl.BlockSpec(memory_space=pltpu.SMEM)
    h1t = pl.pallas_call(
        _gnn_block,
        grid=(BS // BB,),
        in_specs=[
            pl.BlockSpec((BB, J), lambda i: (i, 0)),
            pl.BlockSpec((BB, J), lambda i: (i, 0)),
            pl.BlockSpec((BB, 1), lambda i: (i, 0)),
            pl.BlockSpec((BB, 1), lambda i: (i, 0)),
            pl.BlockSpec((BB, 1), lambda i: (i, 0)),
            pl.BlockSpec((BB, J, N), lambda i: (i, 0, 0)),
            pl.BlockSpec((BB, J, J), lambda i: (i, 0, 0)),
            smem,
            smem,
            pl.BlockSpec((5, H * F0), rep2),
            pl.BlockSpec((H, F0), rep2),
            pl.BlockSpec((H, F0), rep2),
            pl.BlockSpec((H, F0), rep2),
            pl.BlockSpec((1, H * F0), rep2),
            pl.BlockSpec((1, H * F0), rep2),
            pl.BlockSpec((F0, H * ED), rep2),
            pl.BlockSpec((H, ED), rep2),
            pl.BlockSpec((H, ED), rep2),
            pl.BlockSpec((H, ED), rep2),
            pl.BlockSpec((1, H * ED), rep2),
            pl.BlockSpec((1, H * ED), rep2),
        ],
        out_specs=pl.BlockSpec((BB, ED, NP), lambda i: (i, 0, 0)),
        out_shape=jax.ShapeDtypeStruct((BS, ED, NP), jnp.float32),
        scratch_shapes=[
            pltpu.VMEM((8, H * F0), jnp.float32),
            pltpu.VMEM((8, H), jnp.float32),
            pltpu.VMEM((8, H), jnp.float32),
            pltpu.VMEM((F0, H), jnp.float32),
            pltpu.VMEM((F0, H), jnp.float32),
            pltpu.VMEM((F0, H), jnp.float32),
            pltpu.VMEM((F0, H), jnp.float32),
            pltpu.VMEM((ED, H), jnp.float32),
            pltpu.VMEM((ED, H), jnp.float32),
            pltpu.SMEM((2 * H,), jnp.float32),
        ],
    )(norm_h, norm_L, norm_W, norm_P, norm_N, G3, T, ln_g, ln_b,
      W0, al0, ar0, ae0, We0, b0r, W1, al1, ar1, ae1, We1, b1r)

    out = pl.pallas_call(
        _fin_block,
        grid=(NFB,),
        in_specs=[
            pl.BlockSpec((BS, FB, NP), lambda k: (0, k, 0)),
            pl.BlockSpec((N, FB, ED), lambda k: (0, k, 0)),
            pl.BlockSpec((1, ED), lambda k: (0, 0)),
        ],
        out_specs=pl.BlockSpec((BS, ED), lambda k: (0, 0)),
        out_shape=jax.ShapeDtypeStruct((BS, ED), jnp.float32),
    )(h1t, Wl3, blr)
    return out


# speedup vs baseline: 9.6475x; 1.0234x over previous
import jax
import jax.numpy as jnp
from jax.experimental import pallas as pl
from jax.experimental.pallas import tpu as pltpu

J = 100
M = 20
N = J + M
NP = 128
SP = 104
BS = 256
H = 3
F0 = 16
ED = 128
BB = 32
FB = 16
NFB = ED // FB


def _lrelu(x, s):
    return jnp.maximum(x, s * x)


def _mm(x3, w):
    b, n, k = x3.shape
    y = jnp.dot(x3.reshape(b * n, k), w, preferred_element_type=jnp.float32)
    return y.reshape(b, n, -1)


def _gnn_block(nh_ref, nl_ref, nw_ref, np_ref, nn_ref, a_ref, t_ref,
               g_ref, bln_ref, w0_ref, al0_ref, ar0_ref, ae0_ref, we0_ref,
               b0_ref, w1_ref, al1_ref, ar1_ref, ae1_ref, we1_ref, b1_ref,
               h1_ref,
               w0p_s, wel0_s, wer0_s, few0_s, b0c_s,
               wel1_s, wer1_s, few1_s, b1c_s, eec_s):
    @pl.when(pl.program_id(0) == 0)
    def _():
        w0p_s[...] = jnp.concatenate(
            [w0_ref[...], jnp.zeros((3, H * F0), jnp.float32)], axis=0)
        for (w_ref, al_ref, ar_ref, ae_ref, we_ref, b_ref, Fh, Cpad, wel_s,
             wer_s, few_s, bc_s, ebase) in (
                (w0_ref, al0_ref, ar0_ref, ae0_ref, we0_ref, b0_ref, F0, 8,
                 wel0_s, wer0_s, few0_s, b0c_s, 0),
                (w1_ref, al1_ref, ar1_ref, ae1_ref, we1_ref, b1_ref, ED, F0,
                 wel1_s, wer1_s, few1_s, b1c_s, H)):
            C = w_ref.shape[0]
            zpad = jnp.zeros((max(Cpad - C, 1), 1), jnp.float32)
            for h in range(H):
                wsl = w_ref[:, h * Fh:(h + 1) * Fh]
                wel = jnp.sum(wsl * al_ref[h:h + 1, :], axis=1, keepdims=True)
                wer = jnp.sum(wsl * ar_ref[h:h + 1, :], axis=1, keepdims=True)
                wesl = we_ref[:, h * Fh:(h + 1) * Fh]
                if Cpad > C:
                    wel = jnp.concatenate([wel, zpad], axis=0)
                    wer = jnp.concatenate([wer, zpad], axis=0)
                wel_s[:, h:h + 1] = wel
                wer_s[:, h:h + 1] = wer
                few_s[:, h:h + 1] = jnp.swapaxes(wesl, 0, 1)
                bc_s[:, h:h + 1] = jnp.swapaxes(
                    b_ref[:, h * Fh:(h + 1) * Fh], 0, 1)
                eec_s[ebase + h] = jnp.sum(wesl * ae_ref[h:h + 1, :])

    zlane = jnp.zeros((BB, NP - J), jnp.float32)
    lane = jax.lax.broadcasted_iota(jnp.int32, (BB, NP), 1)
    mJ = lane < J
    rows = [jnp.concatenate([nh_ref[...], zlane], axis=1),
            jnp.concatenate([nl_ref[...], zlane], axis=1),
            jnp.where(mJ, nw_ref[...], 0.0),
            jnp.where(mJ, np_ref[...], 0.0),
            jnp.where(mJ, nn_ref[...], 0.0)]
    XT = jnp.stack(rows, axis=1)

    mu = jnp.sum(XT, axis=1, keepdims=True) * (1.0 / 5.0)
    d = XT - mu
    var = jnp.sum(d * d, axis=1, keepdims=True) * (1.0 / 5.0)
    zn = d * jax.lax.rsqrt(var + 1e-5)
    XnT = jnp.concatenate(
        [zn[:, c:c + 1, :] * g_ref[c] + bln_ref[c] for c in range(5)]
        + [jnp.zeros((BB, 3, NP), jnp.float32)], axis=1)

    G = a_ref[...]
    Gp = jnp.concatenate([G, jnp.zeros((BB, SP - J, N), jnp.float32)], axis=1)
    Gp = jnp.concatenate([Gp, jnp.zeros((BB, SP, NP - N), jnp.float32)], axis=2)
    Ab = Gp > 0
    Tr = t_ref[...]
    Tm = jnp.concatenate([Tr, jnp.zeros((BB, SP - J, J), jnp.float32)], axis=1)
    Tm = jnp.concatenate([Tm, jnp.zeros((BB, SP, NP - J), jnp.float32)], axis=2)

    def gat_T(ft, el3, srcT, wer_s, few_s, bc_s, Fh, ebase):
        C = srcT.shape[1]
        acc = None
        for h in range(H):
            fth = ft[:, :, h * Fh:(h + 1) * Fh]
            el = el3[:, :, h:h + 1]
            er = jnp.sum(srcT * wer_s[:, h:h + 1].reshape(1, C, 1),
                         axis=1, keepdims=True)
            eec = eec_s[ebase + h]
            logits = el + er + Tm * eec
            logits = _lrelu(logits, 0.2)
            logits = jnp.where(Ab, logits, -1e9)
            mx = jnp.max(logits, axis=1, keepdims=True)
            ex = jnp.exp(logits - mx)
            den = jnp.sum(ex, axis=1, keepdims=True)
            rden = jnp.where(mx > -1e8, 1.0 / den, 0.0)
            outT = jax.lax.dot_general(
                fth, ex, (((1,), (1,)), ((0,), (0,))),
                preferred_element_type=jnp.float32)
            eaggT = jnp.sum(ex * Tm, axis=1, keepdims=True)
            fewcol = few_s[:, h:h + 1].reshape(1, Fh, 1)
            bcol = bc_s[:, h:h + 1].reshape(1, Fh, 1)
            hh = _lrelu((outT + eaggT * fewcol) * rden + bcol, 0.01)
            acc = hh if acc is None else acc + hh
        return acc * (1.0 / H)

    Xn = jnp.swapaxes(XnT[:, :, :SP], 1, 2)
    ft0 = _mm(Xn, w0p_s[...])
    el03 = _mm(Xn, wel0_s[...])
    h0T = gat_T(ft0, el03, XnT, wer0_s, few0_s, b0c_s, F0, 0)

    h0 = jnp.swapaxes(h0T[:, :, :SP], 1, 2)
    ft1 = _mm(h0, w1_ref[...])
    el13 = _mm(h0, wel1_s[...])
    h1_ref[...] = gat_T(ft1, el13, h0T, wer1_s, few1_s, b1c_s, ED, H)


def _fin_block(h_ref, w_ref, b_ref, o_ref):
    k = pl.program_id(0)
    hblk = h_ref[...]
    wblk = w_ref[...]
    part = None
    for ff in range(FB):
        p = jnp.dot(hblk[:, ff, :N], wblk[:, ff, :],
                    preferred_element_type=jnp.float32)
        part = p if part is None else part + p

    @pl.when(k == 0)
    def _():
        o_ref[...] = part

    @pl.when(k > 0)
    def _():
        o_ref[...] += part

    @pl.when(k == NFB - 1)
    def _():
        o_ref[...] = _lrelu(o_ref[...] + b_ref[...], 0.01)


def kernel(Graph, norm_h, norm_L, norm_W, norm_P, norm_N, T, ln_g, ln_b,
           W0, We0, al0, ar0, ae0, b0, W1, We1, al1, ar1, ae1, b1, Wl, bl):
    G3 = Graph.reshape(BS, J, N)
    b0r = b0.reshape(1, H * F0)
    b1r = b1.reshape(1, H * ED)
    Wl3 = Wl.reshape(N, ED, ED)
    blr = bl.reshape(1, ED)

    rep2 = lambda i: (0, 0)
    smem = pl.BlockSpec(memory_space=pltpu.SMEM)
    h1t = pl.pallas_call(
        _gnn_block,
        grid=(BS // BB,),
        in_specs=[
            pl.BlockSpec((BB, J), lambda i: (i, 0)),
            pl.BlockSpec((BB, J), lambda i: (i, 0)),
            pl.BlockSpec((BB, 1), lambda i: (i, 0)),
            pl.BlockSpec((BB, 1), lambda i: (i, 0)),
            pl.BlockSpec((BB, 1), lambda i: (i, 0)),
            pl.BlockSpec((BB, J, N), lambda i: (i, 0, 0)),
            pl.BlockSpec((BB, J, J), lambda i: (i, 0, 0)),
            smem,
            smem,
            pl.BlockSpec((5, H * F0), rep2),
            pl.BlockSpec((H, F0), rep2),
            pl.BlockSpec((H, F0), rep2),
            pl.BlockSpec((H, F0), rep2),
            pl.BlockSpec((1, H * F0), rep2),
            pl.BlockSpec((1, H * F0), rep2),
            pl.BlockSpec((F0, H * ED), rep2),
            pl.BlockSpec((H, ED), rep2),
            pl.BlockSpec((H, ED), rep2),
            pl.BlockSpec((H, ED), rep2),
            pl.BlockSpec((1, H * ED), rep2),
            pl.BlockSpec((1, H * ED), rep2),
        ],
        out_specs=pl.BlockSpec((BB, ED, NP), lambda i: (i, 0, 0)),
        out_shape=jax.ShapeDtypeStruct((BS, ED, NP), jnp.float32),
        scratch_shapes=[
            pltpu.VMEM((8, H * F0), jnp.float32),
            pltpu.VMEM((8, H), jnp.float32),
            pltpu.VMEM((8, H), jnp.float32),
            pltpu.VMEM((F0, H), jnp.float32),
            pltpu.VMEM((F0, H), jnp.float32),
            pltpu.VMEM((F0, H), jnp.float32),
            pltpu.VMEM((F0, H), jnp.float32),
            pltpu.VMEM((ED, H), jnp.float32),
            pltpu.VMEM((ED, H), jnp.float32),
            pltpu.SMEM((2 * H,), jnp.float32),
        ],
    )(norm_h, norm_L, norm_W, norm_P, norm_N, G3, T, ln_g, ln_b,
      W0, al0, ar0, ae0, We0, b0r, W1, al1, ar1, ae1, We1, b1r)

    out = pl.pallas_call(
        _fin_block,
        grid=(NFB,),
        in_specs=[
            pl.BlockSpec((BS, FB, NP), lambda k: (0, k, 0)),
            pl.BlockSpec((N, FB, ED), lambda k: (0, k, 0)),
            pl.BlockSpec((1, ED), lambda k: (0, 0)),
        ],
        out_specs=pl.BlockSpec((BS, ED), lambda k: (0, 0)),
        out_shape=jax.ShapeDtypeStruct((BS, ED), jnp.float32),
    )(h1t, Wl3, blr)
    return out
